# Initial kernel scaffold; baseline (speedup 1.0000x reference)
import functools
import math

import jax
import jax.numpy as jnp
from jax import lax
from jax.experimental import pallas as pl
from jax.experimental.pallas import tpu as pltpu
from jax.experimental.pallas import tpu_sc as plsc

N = 10000
E = 320000
D = 128
L = 4
ALPHA = 0.1
THETA = 0.5

NP = 10240
NTILES = 32
CH = 128
CPT = 79
E_PAD = NTILES * CPT * CH
ROWS_PER_TILE = NP // 16
NCOPY = ROWS_PER_TILE // CH
BR = 1024

_mesh = plsc.VectorSubcoreMesh(core_axis_name="c", subcore_axis_name="s")


@functools.partial(
    pl.kernel,
    out_type=jax.ShapeDtypeStruct((2, NP, 16), jnp.float32),
    mesh=_mesh,
    scratch_types=[
        pltpu.VMEM((CPT, CH), jnp.int32),
        pltpu.VMEM((CH, 16), jnp.float32),
        pltpu.VMEM((CH, 16), jnp.float32),
        pltpu.VMEM_SHARED((NP, 16), jnp.float32),
    ],
)
def _sc_degree(dst_hbm, out_hbm, dst_v, ones_v, zero_v, acc_sh):
    c = lax.axis_index("c")
    sid = lax.axis_index("s")
    w = c * 16 + sid
    base = sid * ROWS_PER_TILE

    @pl.loop(0, CH)
    def _(r):
        ones_v[r, :] = jnp.full((16,), 1.0, jnp.float32)
        zero_v[r, :] = jnp.zeros((16,), jnp.float32)

    @pl.loop(0, NCOPY)
    def _(j):
        pltpu.sync_copy(zero_v, acc_sh.at[pl.ds(base + j * CH, CH)])

    pltpu.sync_copy(dst_hbm.at[w], dst_v)
    plsc.subcore_barrier()

    @pl.loop(0, CPT)
    def _(j):
        pltpu.sync_copy(ones_v, acc_sh.at[dst_v.at[j]], add=True)

    plsc.subcore_barrier()

    @pl.loop(0, NCOPY)
    def _(j):
        pltpu.sync_copy(acc_sh.at[pl.ds(base + j * CH, CH)],
                        out_hbm.at[c].at[pl.ds(base + j * CH, CH)])


@functools.partial(
    pl.kernel,
    out_type=jax.ShapeDtypeStruct((2, NP, D), jnp.float32),
    mesh=_mesh,
    scratch_types=[
        pltpu.VMEM((CPT, CH), jnp.int32),
        pltpu.VMEM((CPT, CH), jnp.int32),
        pltpu.VMEM((CH, D), jnp.float32),
        pltpu.VMEM_SHARED((NP, D), jnp.float32),
    ],
)
def _sc_propagate(g_hbm, src_hbm, dst_hbm, out_hbm, src_v, dst_v, rows_v,
                  acc_sh):
    c = lax.axis_index("c")
    sid = lax.axis_index("s")
    w = c * 16 + sid
    base = sid * ROWS_PER_TILE

    @pl.loop(0, CH)
    def _(r):
        @pl.loop(0, D // 16)
        def _(q):
            rows_v[r, pl.ds(q * 16, 16)] = jnp.zeros((16,), jnp.float32)

    @pl.loop(0, NCOPY)
    def _(j):
        pltpu.sync_copy(rows_v, acc_sh.at[pl.ds(base + j * CH, CH)])

    pltpu.sync_copy(src_hbm.at[w], src_v)
    pltpu.sync_copy(dst_hbm.at[w], dst_v)
    plsc.subcore_barrier()

    @pl.loop(0, CPT)
    def _(j):
        pltpu.sync_copy(g_hbm.at[src_v.at[j]], rows_v)
        pltpu.sync_copy(rows_v, acc_sh.at[dst_v.at[j]], add=True)

    plsc.subcore_barrier()

    @pl.loop(0, NCOPY)
    def _(j):
        pltpu.sync_copy(acc_sh.at[pl.ds(base + j * CH, CH)],
                        out_hbm.at[c].at[pl.ds(base + j * CH, CH)])


def _dinv_from_deg(deg_ref):
    deg = deg_ref[0, :, 0:1] + deg_ref[1, :, 0:1] + 1.0
    dinv = lax.rsqrt(deg)
    return dinv, 1.0 / deg


def _proj_body(x_ref, w_ref, b_ref, deg_ref, x0_ref, g_ref):
    x0 = jnp.dot(x_ref[...], w_ref[...],
                 preferred_element_type=jnp.float32,
                 precision=lax.Precision.HIGHEST) + b_ref[...]
    dinv, _ = _dinv_from_deg(deg_ref)
    x0_ref[...] = x0
    g_ref[...] = dinv * x0


def _tc_proj(xp, Wp, bp, degp):
    return pl.pallas_call(
        _proj_body,
        grid=(NP // BR,),
        in_specs=[
            pl.BlockSpec((BR, D), lambda i: (i, 0)),
            pl.BlockSpec((D, D), lambda i: (0, 0)),
            pl.BlockSpec((1, D), lambda i: (0, 0)),
            pl.BlockSpec((2, BR, 16), lambda i: (0, i, 0)),
        ],
        out_specs=[pl.BlockSpec((BR, D), lambda i: (i, 0))] * 2,
        out_shape=[jax.ShapeDtypeStruct((NP, D), jnp.float32)] * 2,
    )(xp, Wp, bp.reshape(1, D), degp)


def _make_dense_body(beta, last):
    def body(acc_ref, h_ref, x0_ref, deg_ref, w_ref, *out_refs):
        dinv, dinv2 = _dinv_from_deg(deg_ref)
        h = h_ref[...]
        s = acc_ref[0, :, :] + acc_ref[1, :, :]
        hp = dinv * s + dinv2 * h
        xp = (1.0 - ALPHA) * hp + ALPHA * x0_ref[...]
        raw = (1.0 - beta) * xp + beta * jnp.dot(
            xp, w_ref[...], preferred_element_type=jnp.float32,
            precision=lax.Precision.HIGHEST)
        hn = raw + h
        if last:
            out_refs[0][...] = hn
        else:
            hn = jnp.maximum(hn, 0.0)
            out_refs[0][...] = hn
            out_refs[1][...] = dinv * hn
    return body


def _tc_dense(acc, h, x0p, degp, W, beta, last):
    n_out = 1 if last else 2
    return pl.pallas_call(
        _make_dense_body(beta, last),
        grid=(NP // BR,),
        in_specs=[
            pl.BlockSpec((2, BR, D), lambda i: (0, i, 0)),
            pl.BlockSpec((BR, D), lambda i: (i, 0)),
            pl.BlockSpec((BR, D), lambda i: (i, 0)),
            pl.BlockSpec((2, BR, 16), lambda i: (0, i, 0)),
            pl.BlockSpec((D, D), lambda i: (0, 0)),
        ],
        out_specs=[pl.BlockSpec((BR, D), lambda i: (i, 0))] * n_out,
        out_shape=[jax.ShapeDtypeStruct((NP, D), jnp.float32)] * n_out,
    )(acc, h, x0p, degp, W)


def kernel(x, edge_index, Wp, bp, W_convs):
    src = edge_index[0].astype(jnp.int32)
    dst = edge_index[1].astype(jnp.int32)
    pad = jnp.full((E_PAD - E,), N, dtype=jnp.int32)
    srcp = jnp.concatenate([src, pad]).reshape(NTILES, CPT, CH)
    dstp = jnp.concatenate([dst, pad]).reshape(NTILES, CPT, CH)
    xp = jnp.pad(x, ((0, NP - N), (0, 0)))

    degp = _sc_degree(dstp)
    x0p, g = _tc_proj(xp, Wp, bp, degp)

    h = x0p
    for i in range(L):
        beta = float(math.log(THETA / (i + 1) + 1.0))
        acc = _sc_propagate(g, srcp, dstp)
        if i != L - 1:
            h, g = _tc_dense(acc, h, x0p, degp, W_convs[i], beta, False)
        else:
            (h,) = _tc_dense(acc, h, x0p, degp, W_convs[i], beta, True)
    return h[:N]


# speedup vs baseline: 10.4339x; 10.4339x over previous
import functools
import math

import jax
import jax.numpy as jnp
from jax import lax
from jax.experimental import pallas as pl
from jax.experimental.pallas import tpu as pltpu
from jax.experimental.pallas import tpu_sc as plsc

N = 10000
E = 320000
D = 128
L = 4
ALPHA = 0.1
THETA = 0.5

NP = 10240
NTILES = 32
CH = 128
CPT = 79
E_PAD = NTILES * CPT * CH
ROWS_PER_TILE = NP // 16
NCOPY = ROWS_PER_TILE // CH
BR = 1024

_mesh = plsc.VectorSubcoreMesh(core_axis_name="c", subcore_axis_name="s")


@functools.partial(
    pl.kernel,
    out_type=jax.ShapeDtypeStruct((2, NP, D), jnp.float32),
    mesh=_mesh,
    scratch_types=[
        pltpu.VMEM((CPT, CH), jnp.int32),
        pltpu.VMEM((CH, D), jnp.float32),
        pltpu.VMEM((CH, D), jnp.float32),
        pltpu.VMEM_SHARED((NP, D), jnp.float32),
    ],
)
def _sc_degree(dst_hbm, ones_hbm, zeros_hbm, out_hbm, dst_v, ones_v, zeros_v,
               acc_sh):
    c = lax.axis_index("c")
    sid = lax.axis_index("s")
    w = c * 16 + sid
    base = sid * ROWS_PER_TILE

    pltpu.sync_copy(ones_hbm, ones_v)
    pltpu.sync_copy(zeros_hbm, zeros_v)

    @pl.loop(0, NCOPY)
    def _(j):
        pltpu.sync_copy(zeros_v, acc_sh.at[pl.ds(base + j * CH, CH)])

    pltpu.sync_copy(dst_hbm.at[w], dst_v)
    plsc.subcore_barrier()

    @pl.loop(0, CPT)
    def _(j):
        pltpu.sync_copy(ones_v, acc_sh.at[dst_v.at[j]], add=True)

    plsc.subcore_barrier()

    @pl.loop(0, NCOPY)
    def _(j):
        pltpu.sync_copy(acc_sh.at[pl.ds(base + j * CH, CH)],
                        out_hbm.at[c].at[pl.ds(base + j * CH, CH)])


@functools.partial(
    pl.kernel,
    out_type=jax.ShapeDtypeStruct((2, NP, D), jnp.float32),
    mesh=_mesh,
    scratch_types=[
        pltpu.VMEM((CPT, CH), jnp.int32),
        pltpu.VMEM((CPT, CH), jnp.int32),
        pltpu.VMEM((CH, D), jnp.float32),
        pltpu.VMEM_SHARED((NP, D), jnp.float32),
    ],
)
def _sc_propagate(g_hbm, src_hbm, dst_hbm, zeros_hbm, out_hbm, src_v, dst_v,
                  rows_v, acc_sh):
    c = lax.axis_index("c")
    sid = lax.axis_index("s")
    w = c * 16 + sid
    base = sid * ROWS_PER_TILE

    pltpu.sync_copy(zeros_hbm, rows_v)

    @pl.loop(0, NCOPY)
    def _(j):
        pltpu.sync_copy(rows_v, acc_sh.at[pl.ds(base + j * CH, CH)])

    pltpu.sync_copy(src_hbm.at[w], src_v)
    pltpu.sync_copy(dst_hbm.at[w], dst_v)
    plsc.subcore_barrier()

    @pl.loop(0, CPT)
    def _(j):
        pltpu.sync_copy(g_hbm.at[src_v.at[j]], rows_v)
        pltpu.sync_copy(rows_v, acc_sh.at[dst_v.at[j]], add=True)

    plsc.subcore_barrier()

    @pl.loop(0, NCOPY)
    def _(j):
        pltpu.sync_copy(acc_sh.at[pl.ds(base + j * CH, CH)],
                        out_hbm.at[c].at[pl.ds(base + j * CH, CH)])


def _dinv_from_deg(deg_ref):
    deg = deg_ref[0, :, 0:1] + deg_ref[1, :, 0:1] + 1.0
    dinv = lax.rsqrt(deg)
    return dinv, 1.0 / deg


def _proj_body(x_ref, w_ref, b_ref, deg_ref, x0_ref, g_ref):
    x0 = jnp.dot(x_ref[...], w_ref[...],
                 preferred_element_type=jnp.float32,
                 precision=lax.Precision.HIGHEST) + b_ref[...]
    dinv, _ = _dinv_from_deg(deg_ref)
    x0_ref[...] = x0
    g_ref[...] = dinv * x0


def _tc_proj(xp, Wp, bp, degp):
    return pl.pallas_call(
        _proj_body,
        grid=(NP // BR,),
        in_specs=[
            pl.BlockSpec((BR, D), lambda i: (i, 0)),
            pl.BlockSpec((D, D), lambda i: (0, 0)),
            pl.BlockSpec((1, D), lambda i: (0, 0)),
            pl.BlockSpec((2, BR, D), lambda i: (0, i, 0)),
        ],
        out_specs=[pl.BlockSpec((BR, D), lambda i: (i, 0))] * 2,
        out_shape=[jax.ShapeDtypeStruct((NP, D), jnp.float32)] * 2,
    )(xp, Wp, bp.reshape(1, D), degp)


def _make_dense_body(beta, last):
    def body(acc_ref, h_ref, x0_ref, deg_ref, w_ref, *out_refs):
        dinv, dinv2 = _dinv_from_deg(deg_ref)
        h = h_ref[...]
        s = acc_ref[0, :, :] + acc_ref[1, :, :]
        hp = dinv * s + dinv2 * h
        xp = (1.0 - ALPHA) * hp + ALPHA * x0_ref[...]
        raw = (1.0 - beta) * xp + beta * jnp.dot(
            xp, w_ref[...], preferred_element_type=jnp.float32,
            precision=lax.Precision.HIGHEST)
        hn = raw + h
        if last:
            out_refs[0][...] = hn
        else:
            hn = jnp.maximum(hn, 0.0)
            out_refs[0][...] = hn
            out_refs[1][...] = dinv * hn
    return body


def _tc_dense(acc, h, x0p, degp, W, beta, last):
    n_out = 1 if last else 2
    return pl.pallas_call(
        _make_dense_body(beta, last),
        grid=(NP // BR,),
        in_specs=[
            pl.BlockSpec((2, BR, D), lambda i: (0, i, 0)),
            pl.BlockSpec((BR, D), lambda i: (i, 0)),
            pl.BlockSpec((BR, D), lambda i: (i, 0)),
            pl.BlockSpec((2, BR, D), lambda i: (0, i, 0)),
            pl.BlockSpec((D, D), lambda i: (0, 0)),
        ],
        out_specs=[pl.BlockSpec((BR, D), lambda i: (i, 0))] * n_out,
        out_shape=[jax.ShapeDtypeStruct((NP, D), jnp.float32)] * n_out,
    )(acc, h, x0p, degp, W)


def kernel(x, edge_index, Wp, bp, W_convs):
    src = edge_index[0].astype(jnp.int32)
    dst = edge_index[1].astype(jnp.int32)
    pad = jnp.full((E_PAD - E,), N, dtype=jnp.int32)
    srcp = jnp.concatenate([src, pad]).reshape(NTILES, CPT, CH)
    dstp = jnp.concatenate([dst, pad]).reshape(NTILES, CPT, CH)
    xp = jnp.pad(x, ((0, NP - N), (0, 0)))
    onesD = jnp.ones((CH, D), jnp.float32)
    zerosD = jnp.zeros((CH, D), jnp.float32)

    degp = _sc_degree(dstp, onesD, zerosD)
    x0p, g = _tc_proj(xp, Wp, bp, degp)

    h = x0p
    for i in range(L):
        beta = float(math.log(THETA / (i + 1) + 1.0))
        acc = _sc_propagate(g, srcp, dstp, zerosD)
        if i != L - 1:
            h, g = _tc_dense(acc, h, x0p, degp, W_convs[i], beta, False)
        else:
            (h,) = _tc_dense(acc, h, x0p, degp, W_convs[i], beta, True)
    return h[:N]
